# baseline (speedup 1.0000x reference)
import functools
import jax
import jax.numpy as jnp
from jax.experimental import pallas as pl
from jax.experimental.pallas import tpu as pltpu
from jax.experimental.pallas import tpu_sc as plsc

D = 768
H = 12
E = 8
TOPK = 2
INNER = 3072
N = 2048
DH = D // H

BN = 256
BS = 256
PMAX = TOPK * N + E * BS
NB = PMAX // BS


def _ln(x, g, b):
    mu = jnp.mean(x, axis=-1, keepdims=True)
    var = jnp.mean((x - mu) ** 2, axis=-1, keepdims=True)
    return (x - mu) * jax.lax.rsqrt(var + 1e-5) * g + b


def _bf16_dot(a, b):
    return jax.lax.dot(a.astype(jnp.bfloat16), b.astype(jnp.bfloat16),
                       preferred_element_type=jnp.float32)


def _qkv_kernel(x_ref, g_ref, b_ref, w_ref, o_ref):
    h = _ln(x_ref[...], g_ref[...], b_ref[...])
    o_ref[...] = _bf16_dot(h, w_ref[...])


def _qkv_call(x, g, b, w):
    return pl.pallas_call(
        _qkv_kernel,
        grid=(N // BN,),
        in_specs=[
            pl.BlockSpec((BN, D), lambda i: (i, 0)),
            pl.BlockSpec((1, D), lambda i: (0, 0)),
            pl.BlockSpec((1, D), lambda i: (0, 0)),
            pl.BlockSpec((D, 3 * D), lambda i: (0, 0)),
        ],
        out_specs=pl.BlockSpec((BN, 3 * D), lambda i: (i, 0)),
        out_shape=jax.ShapeDtypeStruct((N, 3 * D), jnp.float32),
    )(x, g, b, w)


def _attn_kernel(thr_ref, q_ref, k_ref, v_ref, o_ref):
    scale = DH ** -0.5
    q = q_ref[0, 0].astype(jnp.bfloat16)
    k = k_ref[0, 0].astype(jnp.bfloat16)
    s = jax.lax.dot_general(q, k, (((1,), (1,)), ((), ())),
                            preferred_element_type=jnp.float32) * scale
    thr = thr_ref[0]
    s = jnp.where(s < thr, jnp.float32(-1e9), s)
    m = jnp.max(s, axis=1, keepdims=True)
    p = jnp.exp(s - m)
    l = jnp.sum(p, axis=1, keepdims=True)
    o = jax.lax.dot(p.astype(jnp.bfloat16), v_ref[0, 0].astype(jnp.bfloat16),
                    preferred_element_type=jnp.float32)
    o_ref[0] = o / l


def _attn_call(qkv3, thr):
    BQ = 256
    return pl.pallas_call(
        _attn_kernel,
        grid=(H, N // BQ),
        in_specs=[
            pl.BlockSpec(memory_space=pltpu.SMEM),
            pl.BlockSpec((1, 1, BQ, DH), lambda h, j: (0, h, j, 0)),
            pl.BlockSpec((1, 1, N, DH), lambda h, j: (1, h, 0, 0)),
            pl.BlockSpec((1, 1, N, DH), lambda h, j: (2, h, 0, 0)),
        ],
        out_specs=pl.BlockSpec((1, BQ, DH), lambda h, j: (h, j, 0)),
        out_shape=jax.ShapeDtypeStruct((H, N, DH), jnp.float32),
    )(thr.reshape(1), qkv3, qkv3, qkv3)


def _proj_kernel(x_ref, a_ref, ow_ref, ob_ref, g2_ref, b2_ref, gw_ref,
                 x2_ref, h2_ref, lg_ref):
    x2 = x_ref[...] + _bf16_dot(a_ref[...], ow_ref[...]) + ob_ref[...]
    x2_ref[...] = x2
    h2 = _ln(x2, g2_ref[...], b2_ref[...])
    h2_ref[...] = h2.astype(jnp.bfloat16)
    lg_ref[...] = _bf16_dot(h2, gw_ref[...])


def _proj_call(x, attn_out, ow, ob, g2, b2, gw_pad):
    return pl.pallas_call(
        _proj_kernel,
        grid=(N // BN,),
        in_specs=[
            pl.BlockSpec((BN, D), lambda i: (i, 0)),
            pl.BlockSpec((BN, D), lambda i: (i, 0)),
            pl.BlockSpec((D, D), lambda i: (0, 0)),
            pl.BlockSpec((1, D), lambda i: (0, 0)),
            pl.BlockSpec((1, D), lambda i: (0, 0)),
            pl.BlockSpec((1, D), lambda i: (0, 0)),
            pl.BlockSpec((D, 128), lambda i: (0, 0)),
        ],
        out_specs=[
            pl.BlockSpec((BN, D), lambda i: (i, 0)),
            pl.BlockSpec((BN, D), lambda i: (i, 0)),
            pl.BlockSpec((BN, 128), lambda i: (i, 0)),
        ],
        out_shape=[
            jax.ShapeDtypeStruct((N, D), jnp.float32),
            jax.ShapeDtypeStruct((N, D), jnp.bfloat16),
            jax.ShapeDtypeStruct((N, 128), jnp.float32),
        ],
    )(x, attn_out, ow, ob, g2, b2, gw_pad)


def _sc_gather(src, idx):
    n_idx = idx.shape[0]
    w = 128
    idx2 = idx.reshape(n_idx // w, w)
    mesh = plsc.VectorSubcoreMesh(core_axis_name="core",
                                  subcore_axis_name="subcore")

    @functools.partial(
        pl.kernel,
        out_type=jax.ShapeDtypeStruct((n_idx, src.shape[1]), src.dtype),
        mesh=mesh)
    def kern(src_hbm, i_hbm, o_hbm):
        def body(i_vmem, o_vmem):
            pltpu.sync_copy(src_hbm.at[i_vmem.at[0]], o_vmem)

        pltpu.emit_pipeline(
            body,
            grid=(n_idx // w,),
            in_specs=[pl.BlockSpec((1, w), lambda i: (i, 0))],
            out_specs=[pl.BlockSpec((w, src.shape[1]), lambda i: (i, 0))],
            core_axis_name=("core", "subcore"),
            dimension_semantics=(pltpu.PARALLEL,),
        )(i_hbm, o_hbm)

    return kern(src, idx2)


def _pack_bf16(a):
    n, d = a.shape
    return jax.lax.bitcast_convert_type(a.reshape(n, d // 2, 2), jnp.int32)


def _unpack_bf16(a):
    n, d2 = a.shape
    return jax.lax.bitcast_convert_type(
        a[..., None], jnp.bfloat16).reshape(n, d2 * 2)


def _sc_gather_bf16(src_bf16, idx):
    return _unpack_bf16(_sc_gather(_pack_bf16(src_bf16), idx))


def _ffn_kernel(be_ref, act_ref, xs_ref, w1_ref, b1_ref, w2_ref,
                b2_ref, eo_ref):
    i = pl.program_id(0)

    @pl.when(act_ref[i] == 1)
    def _():
        h = _bf16_dot(xs_ref[...], w1_ref[0]) + b1_ref[0]
        h = h * 0.5 * (1.0 + jax.lax.erf(h * (2.0 ** -0.5)))
        o = _bf16_dot(h, w2_ref[0]) + b2_ref[0]
        eo_ref[...] = o.astype(jnp.bfloat16)


def _ffn_call(block_expert, block_active, xs, w1, b1, w2, b2):
    grid_spec = pltpu.PrefetchScalarGridSpec(
        num_scalar_prefetch=2,
        grid=(NB,),
        in_specs=[
            pl.BlockSpec((BS, D), lambda i, be, act: (i, 0)),
            pl.BlockSpec((1, D, INNER), lambda i, be, act: (be[i], 0, 0)),
            pl.BlockSpec((1, 1, INNER), lambda i, be, act: (be[i], 0, 0)),
            pl.BlockSpec((1, INNER, D), lambda i, be, act: (be[i], 0, 0)),
            pl.BlockSpec((1, 1, D), lambda i, be, act: (be[i], 0, 0)),
        ],
        out_specs=pl.BlockSpec((BS, D), lambda i, be, act: (i, 0)),
        scratch_shapes=[],
    )
    return pl.pallas_call(
        _ffn_kernel,
        grid_spec=grid_spec,
        out_shape=jax.ShapeDtypeStruct((PMAX, D), jnp.bfloat16),
    )(block_expert, block_active, xs, w1, b1, w2, b2)


def _combine_kernel(x2_ref, g0_ref, g1_ref, w0_ref, w1_ref, o_ref):
    o_ref[...] = (x2_ref[...]
                  + w0_ref[...] * g0_ref[...].astype(jnp.float32)
                  + w1_ref[...] * g1_ref[...].astype(jnp.float32))


def _combine_call(x2, g, w0, w1):
    nb = N // BN
    return pl.pallas_call(
        _combine_kernel,
        grid=(nb,),
        in_specs=[
            pl.BlockSpec((BN, D), lambda j: (j, 0)),
            pl.BlockSpec((BN, D), lambda j: (j, 0)),
            pl.BlockSpec((BN, D), lambda j: (j + nb, 0)),
            pl.BlockSpec((BN, 1), lambda j: (j, 0)),
            pl.BlockSpec((BN, 1), lambda j: (j, 0)),
        ],
        out_specs=pl.BlockSpec((BN, D), lambda j: (j, 0)),
        out_shape=jax.ShapeDtypeStruct((N, D), jnp.float32),
    )(x2, g, g, w0, w1)


def kernel(x, norm1_g, norm1_b, norm2_g, norm2_b, qkv_w, out_w, out_b, thr,
           gating_w, e_w1, e_b1, e_w2, e_b2):
    b, n, d = x.shape
    xf = x.reshape(n, d)

    qkv = _qkv_call(xf, norm1_g.reshape(1, D), norm1_b.reshape(1, D), qkv_w)
    qkv3 = qkv.reshape(N, 3, H, DH).transpose(1, 2, 0, 3)
    attn_h = _attn_call(qkv3, thr)
    attn_out = attn_h.transpose(1, 0, 2).reshape(N, D)
    gw_pad = jnp.pad(gating_w, ((0, 0), (0, 128 - E)))
    x2, h2, lg_pad = _proj_call(xf, attn_out, out_w, out_b.reshape(1, D),
                                norm2_g.reshape(1, D), norm2_b.reshape(1, D),
                                gw_pad)
    logits = lg_pad[:, :E]

    probs = jax.nn.softmax(logits, axis=-1)
    aux = E * jnp.sum(jnp.mean(probs, axis=0) * jnp.sum(probs, axis=0))

    tw, ti = jax.lax.top_k(logits, TOPK)
    tw = jax.nn.softmax(tw, axis=-1)
    eflat = ti.reshape(-1).astype(jnp.int32)
    oh = (eflat[:, None] == jnp.arange(E, dtype=jnp.int32)[None, :])
    oh = oh.astype(jnp.int32)
    csum = jnp.cumsum(oh, axis=0)
    rank = jnp.sum(oh * csum, axis=1) - 1
    counts = csum[-1]
    pcounts = ((counts + BS - 1) // BS) * BS
    poffs = jnp.concatenate([jnp.zeros(1, jnp.int32),
                             jnp.cumsum(pcounts)[:-1].astype(jnp.int32)])
    ptotal = jnp.sum(pcounts)
    ppos = jnp.sum(oh * poffs[None, :], axis=1) + rank
    i_arr = jnp.arange(TOPK * N, dtype=jnp.int32)
    tok_pad = jnp.zeros(PMAX, jnp.int32).at[ppos].set(i_arr // TOPK)
    pos0 = ppos[0::TOPK]
    pos1 = ppos[1::TOPK]
    blk_start = jnp.arange(NB, dtype=jnp.int32) * BS
    block_expert = jnp.clip(
        jnp.searchsorted(poffs, blk_start, side='right').astype(jnp.int32) - 1,
        0, E - 1)
    block_active = (blk_start < ptotal).astype(jnp.int32)

    xs = _sc_gather_bf16(h2, tok_pad)
    eo = _ffn_call(block_expert, block_active, xs, e_w1,
                   e_b1.reshape(E, 1, INNER), e_w2, e_b2.reshape(E, 1, D))
    g = _sc_gather_bf16(eo, jnp.concatenate([pos0, pos1]))
    out = _combine_call(x2, g, tw[:, 0:1], tw[:, 1:2])
    return (out.reshape(b, n, d), aux)


# speedup vs baseline: 1.8465x; 1.8465x over previous
import functools
import jax
import jax.numpy as jnp
from jax.experimental import pallas as pl
from jax.experimental.pallas import tpu as pltpu
from jax.experimental.pallas import tpu_sc as plsc

D = 768
H = 12
E = 8
TOPK = 2
INNER = 3072
N = 2048
DH = D // H

BN = 256
BS = 256
PMAX = TOPK * N + E * BS
NB = PMAX // BS


def _ln(x, g, b):
    mu = jnp.mean(x, axis=-1, keepdims=True)
    var = jnp.mean((x - mu) ** 2, axis=-1, keepdims=True)
    return (x - mu) * jax.lax.rsqrt(var + 1e-5) * g + b


def _bf16_dot(a, b):
    return jax.lax.dot(a.astype(jnp.bfloat16), b.astype(jnp.bfloat16),
                       preferred_element_type=jnp.float32)


def _qkv_kernel(x_ref, g_ref, b_ref, w_ref, o_ref):
    h = _ln(x_ref[...], g_ref[...], b_ref[...])
    o_ref[...] = _bf16_dot(h, w_ref[...])


def _qkv_call(x, g, b, w):
    return pl.pallas_call(
        _qkv_kernel,
        grid=(N // BN,),
        in_specs=[
            pl.BlockSpec((BN, D), lambda i: (i, 0)),
            pl.BlockSpec((1, D), lambda i: (0, 0)),
            pl.BlockSpec((1, D), lambda i: (0, 0)),
            pl.BlockSpec((D, 3 * D), lambda i: (0, 0)),
        ],
        out_specs=pl.BlockSpec((BN, 3 * D), lambda i: (i, 0)),
        out_shape=jax.ShapeDtypeStruct((N, 3 * D), jnp.float32),
    )(x, g, b, w)


def _attn_kernel(thr_ref, q_ref, k_ref, v_ref, o_ref):
    scale = DH ** -0.5
    q = q_ref[0, 0].astype(jnp.bfloat16)
    k = k_ref[0, 0].astype(jnp.bfloat16)
    s = jax.lax.dot_general(q, k, (((1,), (1,)), ((), ())),
                            preferred_element_type=jnp.float32) * scale
    thr = thr_ref[0]
    s = jnp.where(s < thr, jnp.float32(-1e9), s)
    m = jnp.max(s, axis=1, keepdims=True)
    p = jnp.exp(s - m)
    l = jnp.sum(p, axis=1, keepdims=True)
    o = jax.lax.dot(p.astype(jnp.bfloat16), v_ref[0, 0].astype(jnp.bfloat16),
                    preferred_element_type=jnp.float32)
    o_ref[0] = o / l


def _attn_call(qkv3, thr):
    BQ = 256
    return pl.pallas_call(
        _attn_kernel,
        grid=(H, N // BQ),
        in_specs=[
            pl.BlockSpec(memory_space=pltpu.SMEM),
            pl.BlockSpec((1, 1, BQ, DH), lambda h, j: (0, h, j, 0)),
            pl.BlockSpec((1, 1, N, DH), lambda h, j: (1, h, 0, 0)),
            pl.BlockSpec((1, 1, N, DH), lambda h, j: (2, h, 0, 0)),
        ],
        out_specs=pl.BlockSpec((1, BQ, DH), lambda h, j: (h, j, 0)),
        out_shape=jax.ShapeDtypeStruct((H, N, DH), jnp.float32),
    )(thr.reshape(1), qkv3, qkv3, qkv3)


def _proj_kernel(x_ref, a_ref, ow_ref, ob_ref, g2_ref, b2_ref, gw_ref,
                 x2_ref, h2_ref, lg_ref):
    x2 = x_ref[...] + _bf16_dot(a_ref[...], ow_ref[...]) + ob_ref[...]
    x2_ref[...] = x2
    h2 = _ln(x2, g2_ref[...], b2_ref[...])
    h2_ref[...] = h2.astype(jnp.bfloat16)
    lg_ref[...] = _bf16_dot(h2, gw_ref[...])


def _proj_call(x, attn_out, ow, ob, g2, b2, gw_pad):
    return pl.pallas_call(
        _proj_kernel,
        grid=(N // BN,),
        in_specs=[
            pl.BlockSpec((BN, D), lambda i: (i, 0)),
            pl.BlockSpec((BN, D), lambda i: (i, 0)),
            pl.BlockSpec((D, D), lambda i: (0, 0)),
            pl.BlockSpec((1, D), lambda i: (0, 0)),
            pl.BlockSpec((1, D), lambda i: (0, 0)),
            pl.BlockSpec((1, D), lambda i: (0, 0)),
            pl.BlockSpec((D, 128), lambda i: (0, 0)),
        ],
        out_specs=[
            pl.BlockSpec((BN, D), lambda i: (i, 0)),
            pl.BlockSpec((BN, D), lambda i: (i, 0)),
            pl.BlockSpec((BN, 128), lambda i: (i, 0)),
        ],
        out_shape=[
            jax.ShapeDtypeStruct((N, D), jnp.float32),
            jax.ShapeDtypeStruct((N, D), jnp.bfloat16),
            jax.ShapeDtypeStruct((N, 128), jnp.float32),
        ],
    )(x, attn_out, ow, ob, g2, b2, gw_pad)


def _ffn_kernel(be_ref, act_ref, tok_ref, h2_ref, w1_ref, b1_ref, w2_ref,
                b2_ref, eo_ref):
    i = pl.program_id(0)

    @pl.when(act_ref[i] == 1)
    def _():
        iota = jax.lax.broadcasted_iota(jnp.int32, (BS, N), 1)
        oh = (tok_ref[...] == iota).astype(jnp.bfloat16)
        xs = jax.lax.dot(oh, h2_ref[...],
                         preferred_element_type=jnp.float32)
        h = _bf16_dot(xs, w1_ref[0]) + b1_ref[0]
        h = h * 0.5 * (1.0 + jax.lax.erf(h * (2.0 ** -0.5)))
        o = _bf16_dot(h, w2_ref[0]) + b2_ref[0]
        eo_ref[...] = o.astype(jnp.bfloat16)

    @pl.when(act_ref[i] == 0)
    def _():
        eo_ref[...] = jnp.zeros((BS, D), jnp.bfloat16)


def _ffn_call(block_expert, block_active, tok_pad, h2b, w1, b1, w2, b2):
    grid_spec = pltpu.PrefetchScalarGridSpec(
        num_scalar_prefetch=2,
        grid=(NB,),
        in_specs=[
            pl.BlockSpec((BS, 1), lambda i, be, act: (i, 0)),
            pl.BlockSpec((N, D), lambda i, be, act: (0, 0)),
            pl.BlockSpec((1, D, INNER), lambda i, be, act: (be[i], 0, 0)),
            pl.BlockSpec((1, 1, INNER), lambda i, be, act: (be[i], 0, 0)),
            pl.BlockSpec((1, INNER, D), lambda i, be, act: (be[i], 0, 0)),
            pl.BlockSpec((1, 1, D), lambda i, be, act: (be[i], 0, 0)),
        ],
        out_specs=pl.BlockSpec((BS, D), lambda i, be, act: (i, 0)),
        scratch_shapes=[],
    )
    return pl.pallas_call(
        _ffn_kernel,
        grid_spec=grid_spec,
        out_shape=jax.ShapeDtypeStruct((PMAX, D), jnp.bfloat16),
    )(block_expert, block_active, tok_pad, h2b, w1, b1, w2, b2)


def _combine_kernel(x2_ref, eo_ref, p0_ref, p1_ref, w0_ref, w1_ref, o_ref):
    iota = jax.lax.broadcasted_iota(jnp.int32, (BN, PMAX), 1)
    eo = eo_ref[...]
    oh0 = (p0_ref[...] == iota).astype(jnp.bfloat16)
    g0 = jax.lax.dot(oh0, eo, preferred_element_type=jnp.float32)
    oh1 = (p1_ref[...] == iota).astype(jnp.bfloat16)
    g1 = jax.lax.dot(oh1, eo, preferred_element_type=jnp.float32)
    o_ref[...] = x2_ref[...] + w0_ref[...] * g0 + w1_ref[...] * g1


def _combine_call(x2, eo, p0, p1, w0, w1):
    return pl.pallas_call(
        _combine_kernel,
        grid=(N // BN,),
        in_specs=[
            pl.BlockSpec((BN, D), lambda j: (j, 0)),
            pl.BlockSpec((PMAX, D), lambda j: (0, 0)),
            pl.BlockSpec((BN, 1), lambda j: (j, 0)),
            pl.BlockSpec((BN, 1), lambda j: (j, 0)),
            pl.BlockSpec((BN, 1), lambda j: (j, 0)),
            pl.BlockSpec((BN, 1), lambda j: (j, 0)),
        ],
        out_specs=pl.BlockSpec((BN, D), lambda j: (j, 0)),
        out_shape=jax.ShapeDtypeStruct((N, D), jnp.float32),
    )(x2, eo, p0, p1, w0, w1)


def kernel(x, norm1_g, norm1_b, norm2_g, norm2_b, qkv_w, out_w, out_b, thr,
           gating_w, e_w1, e_b1, e_w2, e_b2):
    b, n, d = x.shape
    xf = x.reshape(n, d)

    qkv = _qkv_call(xf, norm1_g.reshape(1, D), norm1_b.reshape(1, D), qkv_w)
    qkv3 = qkv.reshape(N, 3, H, DH).transpose(1, 2, 0, 3)
    attn_h = _attn_call(qkv3, thr)
    attn_out = attn_h.transpose(1, 0, 2).reshape(N, D)
    gw_pad = jnp.pad(gating_w, ((0, 0), (0, 128 - E)))
    x2, h2, lg_pad = _proj_call(xf, attn_out, out_w, out_b.reshape(1, D),
                                norm2_g.reshape(1, D), norm2_b.reshape(1, D),
                                gw_pad)
    logits = lg_pad[:, :E]

    probs = jax.nn.softmax(logits, axis=-1)
    aux = E * jnp.sum(jnp.mean(probs, axis=0) * jnp.sum(probs, axis=0))

    tw, ti = jax.lax.top_k(logits, TOPK)
    tw = jax.nn.softmax(tw, axis=-1)
    eflat = ti.reshape(-1).astype(jnp.int32)
    oh = (eflat[:, None] == jnp.arange(E, dtype=jnp.int32)[None, :])
    oh = oh.astype(jnp.int32)
    csum = jnp.cumsum(oh, axis=0)
    rank = jnp.sum(oh * csum, axis=1) - 1
    counts = csum[-1]
    pcounts = ((counts + BS - 1) // BS) * BS
    poffs = jnp.concatenate([jnp.zeros(1, jnp.int32),
                             jnp.cumsum(pcounts)[:-1].astype(jnp.int32)])
    ptotal = jnp.sum(pcounts)
    ppos = jnp.sum(oh * poffs[None, :], axis=1) + rank
    i_arr = jnp.arange(TOPK * N, dtype=jnp.int32)
    tok_pad = jnp.zeros(PMAX, jnp.int32).at[ppos].set(i_arr // TOPK)
    pos0 = ppos[0::TOPK]
    pos1 = ppos[1::TOPK]
    blk_start = jnp.arange(NB, dtype=jnp.int32) * BS
    block_expert = jnp.clip(
        jnp.searchsorted(poffs, blk_start, side='right').astype(jnp.int32) - 1,
        0, E - 1)
    block_active = (blk_start < ptotal).astype(jnp.int32)

    eo = _ffn_call(block_expert, block_active, tok_pad.reshape(PMAX, 1), h2,
                   e_w1, e_b1.reshape(E, 1, INNER), e_w2,
                   e_b2.reshape(E, 1, D))
    out = _combine_call(x2, eo, pos0.reshape(N, 1), pos1.reshape(N, 1),
                        tw[:, 0:1], tw[:, 1:2])
    return (out.reshape(b, n, d), aux)


# speedup vs baseline: 2.2493x; 1.2181x over previous
import functools
import jax
import jax.numpy as jnp
from jax.experimental import pallas as pl
from jax.experimental.pallas import tpu as pltpu
from jax.experimental.pallas import tpu_sc as plsc

D = 768
H = 12
E = 8
TOPK = 2
INNER = 3072
N = 2048
DH = D // H

BN = 256
BS = 256
PMAX = TOPK * N + E * BS
NB = PMAX // BS


def _ln(x, g, b):
    mu = jnp.mean(x, axis=-1, keepdims=True)
    var = jnp.mean((x - mu) ** 2, axis=-1, keepdims=True)
    return (x - mu) * jax.lax.rsqrt(var + 1e-5) * g + b


def _bf16_dot(a, b):
    return jax.lax.dot(a.astype(jnp.bfloat16), b.astype(jnp.bfloat16),
                       preferred_element_type=jnp.float32)


def _qkv_kernel(x_ref, g_ref, b_ref, w_ref, o_ref):
    h = _ln(x_ref[...], g_ref[...], b_ref[...])
    o_ref[...] = _bf16_dot(h, w_ref[...])


def _qkv_call(x, g, b, w):
    return pl.pallas_call(
        _qkv_kernel,
        grid=(N // BN,),
        in_specs=[
            pl.BlockSpec((BN, D), lambda i: (i, 0)),
            pl.BlockSpec((1, D), lambda i: (0, 0)),
            pl.BlockSpec((1, D), lambda i: (0, 0)),
            pl.BlockSpec((D, 3 * D), lambda i: (0, 0)),
        ],
        out_specs=pl.BlockSpec((BN, 3 * D), lambda i: (i, 0)),
        out_shape=jax.ShapeDtypeStruct((N, 3 * D), jnp.float32),
    )(x, g, b, w)


def _attn_kernel(thr_ref, q_ref, k_ref, v_ref, o_ref):
    scale = DH ** -0.5
    thr = thr_ref[0]
    q = q_ref[...]
    k = k_ref[...]
    v = v_ref[...]
    outs = []
    for hh in range(2):
        sl = slice(hh * DH, (hh + 1) * DH)
        qh = q[:, sl].astype(jnp.bfloat16)
        kh = k[:, sl].astype(jnp.bfloat16)
        vh = v[:, sl].astype(jnp.bfloat16)
        s = jax.lax.dot_general(qh, kh, (((1,), (1,)), ((), ())),
                                preferred_element_type=jnp.float32) * scale
        s = jnp.where(s < thr, jnp.float32(-1e9), s)
        m = jnp.max(s, axis=1, keepdims=True)
        p = jnp.exp(s - m)
        l = jnp.sum(p, axis=1, keepdims=True)
        o = jax.lax.dot(p.astype(jnp.bfloat16), vh,
                        preferred_element_type=jnp.float32)
        outs.append(o / l)
    o_ref[...] = jnp.concatenate(outs, axis=1)


def _attn_call(qkv, thr):
    BQ = 256
    return pl.pallas_call(
        _attn_kernel,
        grid=(H // 2, N // BQ),
        in_specs=[
            pl.BlockSpec(memory_space=pltpu.SMEM),
            pl.BlockSpec((BQ, 2 * DH), lambda g, j: (j, g)),
            pl.BlockSpec((N, 2 * DH), lambda g, j: (0, H // 2 + g)),
            pl.BlockSpec((N, 2 * DH), lambda g, j: (0, H + g)),
        ],
        out_specs=pl.BlockSpec((BQ, 2 * DH), lambda g, j: (j, g)),
        out_shape=jax.ShapeDtypeStruct((N, D), jnp.float32),
    )(thr.reshape(1), qkv, qkv, qkv)


def _proj_kernel(x_ref, a_ref, ow_ref, ob_ref, g2_ref, b2_ref, gw_ref,
                 x2_ref, h2_ref, lg_ref):
    x2 = x_ref[...] + _bf16_dot(a_ref[...], ow_ref[...]) + ob_ref[...]
    x2_ref[...] = x2
    h2 = _ln(x2, g2_ref[...], b2_ref[...])
    h2_ref[...] = h2.astype(jnp.bfloat16)
    lg_ref[...] = _bf16_dot(h2, gw_ref[...])


def _proj_call(x, attn_out, ow, ob, g2, b2, gw_pad):
    return pl.pallas_call(
        _proj_kernel,
        grid=(N // BN,),
        in_specs=[
            pl.BlockSpec((BN, D), lambda i: (i, 0)),
            pl.BlockSpec((BN, D), lambda i: (i, 0)),
            pl.BlockSpec((D, D), lambda i: (0, 0)),
            pl.BlockSpec((1, D), lambda i: (0, 0)),
            pl.BlockSpec((1, D), lambda i: (0, 0)),
            pl.BlockSpec((1, D), lambda i: (0, 0)),
            pl.BlockSpec((D, 128), lambda i: (0, 0)),
        ],
        out_specs=[
            pl.BlockSpec((BN, D), lambda i: (i, 0)),
            pl.BlockSpec((BN, D), lambda i: (i, 0)),
            pl.BlockSpec((BN, 128), lambda i: (i, 0)),
        ],
        out_shape=[
            jax.ShapeDtypeStruct((N, D), jnp.float32),
            jax.ShapeDtypeStruct((N, D), jnp.bfloat16),
            jax.ShapeDtypeStruct((N, 128), jnp.float32),
        ],
    )(x, attn_out, ow, ob, g2, b2, gw_pad)


def _ffn_kernel(be_ref, act_ref, tok_ref, h2_ref, w1_ref, b1_ref, w2_ref,
                b2_ref, eo_ref):
    i = pl.program_id(0)

    @pl.when(act_ref[i] == 1)
    def _():
        iota = jax.lax.broadcasted_iota(jnp.int32, (BS, N), 1)
        oh = (tok_ref[...] == iota).astype(jnp.bfloat16)
        xs = jax.lax.dot(oh, h2_ref[...],
                         preferred_element_type=jnp.float32)
        h = _bf16_dot(xs, w1_ref[0]) + b1_ref[0]
        h = h * 0.5 * (1.0 + jax.lax.erf(h * (2.0 ** -0.5)))
        o = _bf16_dot(h, w2_ref[0]) + b2_ref[0]
        eo_ref[...] = o.astype(jnp.bfloat16)

    @pl.when(act_ref[i] == 0)
    def _():
        eo_ref[...] = jnp.zeros((BS, D), jnp.bfloat16)


def _ffn_call(block_expert, block_active, tok_pad, h2b, w1, b1, w2, b2):
    grid_spec = pltpu.PrefetchScalarGridSpec(
        num_scalar_prefetch=2,
        grid=(NB,),
        in_specs=[
            pl.BlockSpec((BS, 1), lambda i, be, act: (i, 0)),
            pl.BlockSpec((N, D), lambda i, be, act: (0, 0)),
            pl.BlockSpec((1, D, INNER), lambda i, be, act: (be[i], 0, 0)),
            pl.BlockSpec((1, 1, INNER), lambda i, be, act: (be[i], 0, 0)),
            pl.BlockSpec((1, INNER, D), lambda i, be, act: (be[i], 0, 0)),
            pl.BlockSpec((1, 1, D), lambda i, be, act: (be[i], 0, 0)),
        ],
        out_specs=pl.BlockSpec((BS, D), lambda i, be, act: (i, 0)),
        scratch_shapes=[],
    )
    return pl.pallas_call(
        _ffn_kernel,
        grid_spec=grid_spec,
        out_shape=jax.ShapeDtypeStruct((PMAX, D), jnp.bfloat16),
    )(block_expert, block_active, tok_pad, h2b, w1, b1, w2, b2)


def _combine_kernel(x2_ref, eo_ref, p0_ref, p1_ref, w0_ref, w1_ref, o_ref):
    iota = jax.lax.broadcasted_iota(jnp.int32, (BN, PMAX), 1)
    eo = eo_ref[...]
    oh0 = (p0_ref[...] == iota).astype(jnp.bfloat16)
    g0 = jax.lax.dot(oh0, eo, preferred_element_type=jnp.float32)
    oh1 = (p1_ref[...] == iota).astype(jnp.bfloat16)
    g1 = jax.lax.dot(oh1, eo, preferred_element_type=jnp.float32)
    o_ref[...] = x2_ref[...] + w0_ref[...] * g0 + w1_ref[...] * g1


def _combine_call(x2, eo, p0, p1, w0, w1):
    return pl.pallas_call(
        _combine_kernel,
        grid=(N // BN,),
        in_specs=[
            pl.BlockSpec((BN, D), lambda j: (j, 0)),
            pl.BlockSpec((PMAX, D), lambda j: (0, 0)),
            pl.BlockSpec((BN, 1), lambda j: (j, 0)),
            pl.BlockSpec((BN, 1), lambda j: (j, 0)),
            pl.BlockSpec((BN, 1), lambda j: (j, 0)),
            pl.BlockSpec((BN, 1), lambda j: (j, 0)),
        ],
        out_specs=pl.BlockSpec((BN, D), lambda j: (j, 0)),
        out_shape=jax.ShapeDtypeStruct((N, D), jnp.float32),
    )(x2, eo, p0, p1, w0, w1)


def kernel(x, norm1_g, norm1_b, norm2_g, norm2_b, qkv_w, out_w, out_b, thr,
           gating_w, e_w1, e_b1, e_w2, e_b2):
    b, n, d = x.shape
    xf = x.reshape(n, d)

    qkv = _qkv_call(xf, norm1_g.reshape(1, D), norm1_b.reshape(1, D), qkv_w)
    attn_out = _attn_call(qkv, thr)
    gw_pad = jnp.pad(gating_w, ((0, 0), (0, 128 - E)))
    x2, h2, lg_pad = _proj_call(xf, attn_out, out_w, out_b.reshape(1, D),
                                norm2_g.reshape(1, D), norm2_b.reshape(1, D),
                                gw_pad)
    logits = lg_pad[:, :E]

    probs = jax.nn.softmax(logits, axis=-1)
    aux = E * jnp.sum(jnp.mean(probs, axis=0) * jnp.sum(probs, axis=0))

    tw, ti = jax.lax.top_k(logits, TOPK)
    tw = jax.nn.softmax(tw, axis=-1)
    eflat = ti.reshape(-1).astype(jnp.int32)
    oh = (eflat[:, None] == jnp.arange(E, dtype=jnp.int32)[None, :])
    oh = oh.astype(jnp.int32)
    csum = jnp.cumsum(oh, axis=0)
    rank = jnp.sum(oh * csum, axis=1) - 1
    counts = csum[-1]
    pcounts = ((counts + BS - 1) // BS) * BS
    poffs = jnp.concatenate([jnp.zeros(1, jnp.int32),
                             jnp.cumsum(pcounts)[:-1].astype(jnp.int32)])
    ptotal = jnp.sum(pcounts)
    ppos = jnp.sum(oh * poffs[None, :], axis=1) + rank
    i_arr = jnp.arange(TOPK * N, dtype=jnp.int32)
    tok_pad = jnp.zeros(PMAX, jnp.int32).at[ppos].set(i_arr // TOPK)
    pos0 = ppos[0::TOPK]
    pos1 = ppos[1::TOPK]
    blk_start = jnp.arange(NB, dtype=jnp.int32) * BS
    block_expert = jnp.clip(
        jnp.searchsorted(poffs, blk_start, side='right').astype(jnp.int32) - 1,
        0, E - 1)
    block_active = (blk_start < ptotal).astype(jnp.int32)

    eo = _ffn_call(block_expert, block_active, tok_pad.reshape(PMAX, 1), h2,
                   e_w1, e_b1.reshape(E, 1, INNER), e_w2,
                   e_b2.reshape(E, 1, D))
    out = _combine_call(x2, eo, pos0.reshape(N, 1), pos1.reshape(N, 1),
                        tw[:, 0:1], tw[:, 1:2])
    return (out.reshape(b, n, d), aux)


# speedup vs baseline: 2.3917x; 1.0633x over previous
import functools
import jax
import jax.numpy as jnp
from jax.experimental import pallas as pl
from jax.experimental.pallas import tpu as pltpu
from jax.experimental.pallas import tpu_sc as plsc

D = 768
H = 12
E = 8
TOPK = 2
INNER = 3072
N = 2048
DH = D // H

BN = 256
BS = 256
PMAX = TOPK * N + E * BS
NB = PMAX // BS


def _ln(x, g, b):
    mu = jnp.mean(x, axis=-1, keepdims=True)
    var = jnp.mean((x - mu) ** 2, axis=-1, keepdims=True)
    return (x - mu) * jax.lax.rsqrt(var + 1e-5) * g + b


def _bf16_dot(a, b):
    return jax.lax.dot(a.astype(jnp.bfloat16), b.astype(jnp.bfloat16),
                       preferred_element_type=jnp.float32)


def _qkv_kernel(x_ref, g_ref, b_ref, w_ref, o_ref):
    h = _ln(x_ref[...], g_ref[...], b_ref[...])
    o_ref[...] = _bf16_dot(h, w_ref[...])


def _qkv_call(x, g, b, w):
    return pl.pallas_call(
        _qkv_kernel,
        grid=(N // BN,),
        in_specs=[
            pl.BlockSpec((BN, D), lambda i: (i, 0)),
            pl.BlockSpec((1, D), lambda i: (0, 0)),
            pl.BlockSpec((1, D), lambda i: (0, 0)),
            pl.BlockSpec((D, 3 * D), lambda i: (0, 0)),
        ],
        out_specs=pl.BlockSpec((BN, 3 * D), lambda i: (i, 0)),
        out_shape=jax.ShapeDtypeStruct((N, 3 * D), jnp.float32),
    )(x, g, b, w)


def _attn_kernel(thr_ref, q_ref, k_ref, v_ref, o_ref):
    scale = DH ** -0.5
    thr = thr_ref[0]
    q = q_ref[...]
    k = k_ref[...]
    v = v_ref[...]
    outs = []
    for hh in range(2):
        sl = slice(hh * DH, (hh + 1) * DH)
        qh = q[:, sl].astype(jnp.bfloat16)
        kh = k[:, sl].astype(jnp.bfloat16)
        vh = v[:, sl].astype(jnp.bfloat16)
        s = jax.lax.dot_general(qh, kh, (((1,), (1,)), ((), ())),
                                preferred_element_type=jnp.float32) * scale
        s = jnp.where(s < thr, jnp.float32(-1e9), s)
        m = jnp.max(s, axis=1, keepdims=True)
        p = jnp.exp(s - m)
        l = jnp.sum(p, axis=1, keepdims=True)
        o = jax.lax.dot(p.astype(jnp.bfloat16), vh,
                        preferred_element_type=jnp.float32)
        outs.append(o / l)
    o_ref[...] = jnp.concatenate(outs, axis=1)


def _attn_call(qkv, thr):
    BQ = 512
    return pl.pallas_call(
        _attn_kernel,
        grid=(H // 2, N // BQ),
        in_specs=[
            pl.BlockSpec(memory_space=pltpu.SMEM),
            pl.BlockSpec((BQ, 2 * DH), lambda g, j: (j, g)),
            pl.BlockSpec((N, 2 * DH), lambda g, j: (0, H // 2 + g)),
            pl.BlockSpec((N, 2 * DH), lambda g, j: (0, H + g)),
        ],
        out_specs=pl.BlockSpec((BQ, 2 * DH), lambda g, j: (j, g)),
        out_shape=jax.ShapeDtypeStruct((N, D), jnp.float32),
    )(thr.reshape(1), qkv, qkv, qkv)


def _proj_kernel(x_ref, a_ref, ow_ref, ob_ref, g2_ref, b2_ref, gw_ref,
                 x2_ref, h2_ref, lg_ref):
    x2 = x_ref[...] + _bf16_dot(a_ref[...], ow_ref[...]) + ob_ref[...]
    x2_ref[...] = x2
    h2 = _ln(x2, g2_ref[...], b2_ref[...])
    h2_ref[...] = h2.astype(jnp.bfloat16)
    lg_ref[...] = _bf16_dot(h2, gw_ref[...])


def _proj_call(x, attn_out, ow, ob, g2, b2, gw_pad):
    return pl.pallas_call(
        _proj_kernel,
        grid=(N // BN,),
        in_specs=[
            pl.BlockSpec((BN, D), lambda i: (i, 0)),
            pl.BlockSpec((BN, D), lambda i: (i, 0)),
            pl.BlockSpec((D, D), lambda i: (0, 0)),
            pl.BlockSpec((1, D), lambda i: (0, 0)),
            pl.BlockSpec((1, D), lambda i: (0, 0)),
            pl.BlockSpec((1, D), lambda i: (0, 0)),
            pl.BlockSpec((D, 128), lambda i: (0, 0)),
        ],
        out_specs=[
            pl.BlockSpec((BN, D), lambda i: (i, 0)),
            pl.BlockSpec((BN, D), lambda i: (i, 0)),
            pl.BlockSpec((BN, 128), lambda i: (i, 0)),
        ],
        out_shape=[
            jax.ShapeDtypeStruct((N, D), jnp.float32),
            jax.ShapeDtypeStruct((N, D), jnp.bfloat16),
            jax.ShapeDtypeStruct((N, 128), jnp.float32),
        ],
    )(x, attn_out, ow, ob, g2, b2, gw_pad)


def _ffn_kernel(be_ref, act_ref, tok_ref, h2_ref, w1_ref, b1_ref, w2_ref,
                b2_ref, eo_ref):
    i = pl.program_id(0)

    @pl.when(act_ref[i] == 1)
    def _():
        iota = jax.lax.broadcasted_iota(jnp.int32, (BS, N), 1)
        oh = (tok_ref[...] == iota).astype(jnp.bfloat16)
        xs = jax.lax.dot(oh, h2_ref[...],
                         preferred_element_type=jnp.float32)
        h = _bf16_dot(xs, w1_ref[0]) + b1_ref[0]
        h = h * 0.5 * (1.0 + jax.lax.erf(h * (2.0 ** -0.5)))
        o = _bf16_dot(h, w2_ref[0]) + b2_ref[0]
        eo_ref[...] = o.astype(jnp.bfloat16)

    @pl.when(act_ref[i] == 0)
    def _():
        eo_ref[...] = jnp.zeros((BS, D), jnp.bfloat16)


def _ffn_call(block_expert, block_active, tok_pad, h2b, w1, b1, w2, b2):
    grid_spec = pltpu.PrefetchScalarGridSpec(
        num_scalar_prefetch=2,
        grid=(NB,),
        in_specs=[
            pl.BlockSpec((BS, 1), lambda i, be, act: (i, 0)),
            pl.BlockSpec((N, D), lambda i, be, act: (0, 0)),
            pl.BlockSpec((1, D, INNER), lambda i, be, act: (be[i], 0, 0)),
            pl.BlockSpec((1, 1, INNER), lambda i, be, act: (be[i], 0, 0)),
            pl.BlockSpec((1, INNER, D), lambda i, be, act: (be[i], 0, 0)),
            pl.BlockSpec((1, 1, D), lambda i, be, act: (be[i], 0, 0)),
        ],
        out_specs=pl.BlockSpec((BS, D), lambda i, be, act: (i, 0)),
        scratch_shapes=[],
    )
    return pl.pallas_call(
        _ffn_kernel,
        grid_spec=grid_spec,
        out_shape=jax.ShapeDtypeStruct((PMAX, D), jnp.bfloat16),
    )(block_expert, block_active, tok_pad, h2b, w1, b1, w2, b2)


def _combine_kernel(x2_ref, eo_ref, p0_ref, p1_ref, w0_ref, w1_ref, o_ref):
    iota = jax.lax.broadcasted_iota(jnp.int32, (BN, PMAX), 1)
    comb = jnp.where(p0_ref[...] == iota, w0_ref[...], 0.0)
    comb = jnp.where(p1_ref[...] == iota, w1_ref[...], comb)
    g = jax.lax.dot(comb.astype(jnp.bfloat16), eo_ref[...],
                    preferred_element_type=jnp.float32)
    o_ref[...] = x2_ref[...] + g


def _combine_call(x2, eo, p0, p1, w0, w1):
    return pl.pallas_call(
        _combine_kernel,
        grid=(N // BN,),
        in_specs=[
            pl.BlockSpec((BN, D), lambda j: (j, 0)),
            pl.BlockSpec((PMAX, D), lambda j: (0, 0)),
            pl.BlockSpec((BN, 1), lambda j: (j, 0)),
            pl.BlockSpec((BN, 1), lambda j: (j, 0)),
            pl.BlockSpec((BN, 1), lambda j: (j, 0)),
            pl.BlockSpec((BN, 1), lambda j: (j, 0)),
        ],
        out_specs=pl.BlockSpec((BN, D), lambda j: (j, 0)),
        out_shape=jax.ShapeDtypeStruct((N, D), jnp.float32),
    )(x2, eo, p0, p1, w0, w1)


def kernel(x, norm1_g, norm1_b, norm2_g, norm2_b, qkv_w, out_w, out_b, thr,
           gating_w, e_w1, e_b1, e_w2, e_b2):
    b, n, d = x.shape
    xf = x.reshape(n, d)

    qkv = _qkv_call(xf, norm1_g.reshape(1, D), norm1_b.reshape(1, D), qkv_w)
    attn_out = _attn_call(qkv, thr)
    gw_pad = jnp.pad(gating_w, ((0, 0), (0, 128 - E)))
    x2, h2, lg_pad = _proj_call(xf, attn_out, out_w, out_b.reshape(1, D),
                                norm2_g.reshape(1, D), norm2_b.reshape(1, D),
                                gw_pad)
    logits = lg_pad[:, :E]

    probs = jax.nn.softmax(logits, axis=-1)
    aux = E * jnp.sum(jnp.mean(probs, axis=0) * jnp.sum(probs, axis=0))

    tw, ti = jax.lax.top_k(logits, TOPK)
    tw = jax.nn.softmax(tw, axis=-1)
    eflat = ti.reshape(-1).astype(jnp.int32)
    oh = (eflat[:, None] == jnp.arange(E, dtype=jnp.int32)[None, :])
    oh = oh.astype(jnp.int32)
    csum = jnp.cumsum(oh, axis=0)
    rank = jnp.sum(oh * csum, axis=1) - 1
    counts = csum[-1]
    pcounts = ((counts + BS - 1) // BS) * BS
    poffs = jnp.concatenate([jnp.zeros(1, jnp.int32),
                             jnp.cumsum(pcounts)[:-1].astype(jnp.int32)])
    ptotal = jnp.sum(pcounts)
    ppos = jnp.sum(oh * poffs[None, :], axis=1) + rank
    i_arr = jnp.arange(TOPK * N, dtype=jnp.int32)
    tok_pad = jnp.zeros(PMAX, jnp.int32).at[ppos].set(i_arr // TOPK)
    pos0 = ppos[0::TOPK]
    pos1 = ppos[1::TOPK]
    blk_start = jnp.arange(NB, dtype=jnp.int32) * BS
    block_expert = jnp.clip(
        jnp.searchsorted(poffs, blk_start, side='right').astype(jnp.int32) - 1,
        0, E - 1)
    block_active = (blk_start < ptotal).astype(jnp.int32)

    eo = _ffn_call(block_expert, block_active, tok_pad.reshape(PMAX, 1), h2,
                   e_w1, e_b1.reshape(E, 1, INNER), e_w2,
                   e_b2.reshape(E, 1, D))
    out = _combine_call(x2, eo, pos0.reshape(N, 1), pos1.reshape(N, 1),
                        tw[:, 0:1], tw[:, 1:2])
    return (out.reshape(b, n, d), aux)
